# baseline (speedup 1.0000x reference)
import functools

import jax
import jax.numpy as jnp
from jax import lax
from jax.experimental import pallas as pl
from jax.experimental.pallas import tpu as pltpu
from jax.experimental.pallas import tpu_sc as plsc

NUM_FEATURES = 100000
EMBED_DIM = 128
BATCH = 4096
N_FIELDS = 26

_INFO = plsc.get_sparse_core_info()
_NC = _INFO.num_cores
_NS = _INFO.num_subcores
_NW = _NC * _NS

_N_SLICES = 2
_SLICE = BATCH // _N_SLICES
_BATCH_PER_W = _SLICE // _NW
_BPC = 4
_ROWS_PER_C = _BPC * N_FIELDS
_N_CHUNKS = _BATCH_PER_W // _BPC
_NB = 8
_LA = 4


def _sc_gather(idx, table):
    mesh = plsc.VectorSubcoreMesh(core_axis_name="c", subcore_axis_name="s")

    @functools.partial(
        pl.kernel,
        out_type=jax.ShapeDtypeStruct((_SLICE, N_FIELDS, EMBED_DIM), jnp.float32),
        mesh=mesh,
        scratch_types=(
            [pltpu.VMEM((_BATCH_PER_W * N_FIELDS,), jnp.int32)]
            + [pltpu.VMEM((_ROWS_PER_C, EMBED_DIM), jnp.float32)] * _NB
            + [pltpu.SemaphoreType.DMA] * (2 * _NB)
        ),
    )
    def k(idx_hbm, table_hbm, out_hbm, idx_v, *rest):
        bufs = rest[:_NB]
        gsems = rest[_NB:2 * _NB]
        osems = rest[2 * _NB:]

        wid = lax.axis_index("s") * _NC + lax.axis_index("c")
        base = wid * _BATCH_PER_W
        pltpu.sync_copy(idx_hbm.at[wid], idx_v)

        def gather_start(c, b):
            pltpu.async_copy(
                table_hbm.at[idx_v.at[pl.ds(c * _ROWS_PER_C, _ROWS_PER_C)]],
                bufs[b], gsems[b],
            )

        def gather_wait(b):
            pltpu.make_async_copy(
                table_hbm.at[idx_v.at[pl.ds(0, _ROWS_PER_C)]], bufs[b], gsems[b]
            ).wait()

        def out_start(c, b):
            for j in range(_BPC):
                pltpu.async_copy(
                    bufs[b].at[pl.ds(j * N_FIELDS, N_FIELDS)],
                    out_hbm.at[base + c * _BPC + j],
                    osems[b],
                )

        def out_wait(b):
            for _ in range(_BPC):
                pltpu.make_async_copy(
                    bufs[b].at[pl.ds(0, N_FIELDS)], out_hbm.at[base], osems[b]
                ).wait()

        for c in range(_LA):
            gather_start(c, c)

        def body(p, carry):
            for b in range(_NB):
                c = p * _NB + b
                gather_wait(b)
                out_start(c, b)
                bn = (b + _LA) % _NB
                if b < _LA:
                    @pl.when(p >= 1)
                    def _():
                        out_wait(bn)
                    gather_start(c + _LA, bn)
                else:
                    @pl.when(p < _N_CHUNKS // _NB - 1)
                    def _():
                        out_wait(bn)
                        gather_start(c + _LA, bn)
            return carry

        lax.fori_loop(0, _N_CHUNKS // _NB, body, 0)

        for b in range(_NB):
            out_wait(b)

    return k(idx, table)


def kernel(X, table):
    Xi = X.astype(jnp.int32)
    outs = []
    for s in range(_N_SLICES):
        Xs = Xi[s * _SLICE:(s + 1) * _SLICE]
        idx = Xs.reshape(_NW, _BATCH_PER_W * N_FIELDS)
        outs.append(_sc_gather(idx, table))
    return jnp.concatenate(outs, axis=0)


# speedup vs baseline: 2.6087x; 2.6087x over previous
import functools

import jax
import jax.numpy as jnp
from jax import lax
from jax.experimental import pallas as pl
from jax.experimental.pallas import tpu as pltpu
from jax.experimental.pallas import tpu_sc as plsc

NUM_FEATURES = 100000
EMBED_DIM = 128
BATCH = 4096
N_FIELDS = 26

_INFO = plsc.get_sparse_core_info()
_NC = _INFO.num_cores
_NS = _INFO.num_subcores
_NW = _NC * _NS

_BATCH_PER_W = BATCH // _NW
_NB = 4
_LA = 2


def _sc_gather(idx, table):
    mesh = plsc.VectorSubcoreMesh(core_axis_name="c", subcore_axis_name="s")

    @functools.partial(
        pl.kernel,
        out_type=jax.ShapeDtypeStruct((N_FIELDS * BATCH, EMBED_DIM), jnp.float32),
        mesh=mesh,
        scratch_types=(
            [pltpu.VMEM((N_FIELDS, _BATCH_PER_W), jnp.int32)]
            + [pltpu.VMEM((_BATCH_PER_W, EMBED_DIM), jnp.float32)] * _NB
            + [pltpu.SemaphoreType.DMA] * (2 * _NB)
        ),
    )
    def k(idx_hbm, table_hbm, out_hbm, idx_v, *rest):
        bufs = rest[:_NB]
        gsems = rest[_NB:2 * _NB]
        osems = rest[2 * _NB:]

        wid = lax.axis_index("s") * _NC + lax.axis_index("c")
        base = wid * _BATCH_PER_W
        pltpu.sync_copy(idx_hbm.at[wid], idx_v)

        def gather_start(f, b):
            pltpu.async_copy(table_hbm.at[idx_v.at[f]], bufs[b], gsems[b])

        def gather_wait(b):
            pltpu.make_async_copy(
                table_hbm.at[idx_v.at[0]], bufs[b], gsems[b]
            ).wait()

        def out_start(f, b):
            pltpu.async_copy(
                bufs[b],
                out_hbm.at[pl.ds(f * BATCH + base, _BATCH_PER_W)],
                osems[b],
            )

        def out_wait(b):
            pltpu.make_async_copy(
                bufs[b], out_hbm.at[pl.ds(base, _BATCH_PER_W)], osems[b]
            ).wait()

        out_pending = [False] * _NB

        def drain_out(b):
            if out_pending[b]:
                out_wait(b)
                out_pending[b] = False

        for f in range(min(_LA, N_FIELDS)):
            gather_start(f, f % _NB)
        for f in range(N_FIELDS):
            b = f % _NB
            gather_wait(b)
            out_start(f, b)
            out_pending[b] = True
            nf = f + _LA
            if nf < N_FIELDS:
                bn = nf % _NB
                drain_out(bn)
                gather_start(nf, bn)
        for b in range(_NB):
            drain_out(b)

    return k(idx, table)


def kernel(X, table):
    idx = X.astype(jnp.int32).reshape(_NW, _BATCH_PER_W, N_FIELDS)
    idx = idx.transpose(0, 2, 1)
    out = _sc_gather(idx, table)
    return out.reshape(N_FIELDS, BATCH, EMBED_DIM).transpose(1, 0, 2)
